# Initial kernel scaffold; baseline (speedup 1.0000x reference)
import jax
import jax.numpy as jnp
from jax.experimental import pallas as pl


def kernel(x, W1, b1, W2, b2, W3, b3):
    raise NotImplementedError("write your pallas kernel here")


# speedup vs baseline: 16.6688x; 16.6688x over previous
import functools

import jax
import jax.numpy as jnp
from jax.experimental import pallas as pl
from jax.experimental.pallas import tpu as pltpu

_B, _C, _N = 8, 32, 2048
_OUT, _SR, _K = 64, 2, 4
_TN = 256
_NT = _N // _TN

_F32 = jnp.float32


def _body(x_full_ref, x_tile_ref, W1_ref, b1_ref, W2_ref, b2_ref,
          W3_ref, b3_ref, out0_ref, out1_ref):
  xb = x_full_ref[0]
  xt = x_tile_ref[0]

  col_n2 = jnp.sum(xb * xb, axis=0, keepdims=True)
  row_n2 = jnp.sum(xt * xt, axis=0)[:, None]
  prod = jax.lax.dot_general(xt, xb, (((0,), (0,)), ((), ())),
                             preferred_element_type=_F32)
  inner = -2.0 * prod
  dist = -col_n2 - inner - row_n2

  iota = jax.lax.broadcasted_iota(jnp.int32, (_TN, _N), 1)

  acc0 = None
  acc1 = None
  for _ in range(_K):
    m = jnp.max(dist, axis=1, keepdims=True)
    idx = jnp.min(jnp.where(dist == m, iota, _N), axis=1, keepdims=True)
    sel = iota == idx
    onehot = sel.astype(_F32)
    xj = jax.lax.dot_general(xb, onehot, (((1,), (1,)), ((), ())),
                             preferred_element_type=_F32)
    dist = jnp.where(sel, -jnp.inf, dist)

    feat = jnp.concatenate([xt, xj], axis=0)
    e1 = jax.lax.dot_general(W1_ref[...], feat, (((1,), (0,)), ((), ())),
                             preferred_element_type=_F32)
    e1 = e1 + b1_ref[...]
    g = jax.nn.relu(jnp.concatenate([e1, feat], axis=0))
    e2 = jax.lax.dot_general(W2_ref[...], g, (((1,), (0,)), ((), ())),
                             preferred_element_type=_F32)
    e2 = jax.nn.relu(e2 + b2_ref[...])
    h0 = jax.lax.dot_general(W3_ref[...], e2[:_OUT], (((1,), (0,)), ((), ())),
                             preferred_element_type=_F32)
    h1 = jax.lax.dot_general(W3_ref[...], e2[_OUT:], (((1,), (0,)), ((), ())),
                             preferred_element_type=_F32)
    h0 = h0 + b3_ref[...]
    h1 = h1 + b3_ref[...]
    acc0 = h0 if acc0 is None else jnp.maximum(acc0, h0)
    acc1 = h1 if acc1 is None else jnp.maximum(acc1, h1)

  out0_ref[0] = acc0
  out1_ref[0] = acc1


@jax.jit
def kernel(x, W1, b1, W2, b2, W3, b3):
  b1c = b1.reshape(_OUT, 1)
  b2c = b2.reshape(_OUT * _SR, 1)
  b3c = b3.reshape(_OUT, 1)

  out0, out1 = pl.pallas_call(
      _body,
      grid=(_B, _NT),
      in_specs=[
          pl.BlockSpec((1, _C, _N), lambda b, t: (b, 0, 0)),
          pl.BlockSpec((1, _C, _TN), lambda b, t: (b, 0, t)),
          pl.BlockSpec((_OUT, 2 * _C), lambda b, t: (0, 0)),
          pl.BlockSpec((_OUT, 1), lambda b, t: (0, 0)),
          pl.BlockSpec((_OUT * _SR, 2 * _C + _OUT), lambda b, t: (0, 0)),
          pl.BlockSpec((_OUT * _SR, 1), lambda b, t: (0, 0)),
          pl.BlockSpec((_OUT, _OUT), lambda b, t: (0, 0)),
          pl.BlockSpec((_OUT, 1), lambda b, t: (0, 0)),
      ],
      out_specs=[
          pl.BlockSpec((1, _OUT, _TN), lambda b, t: (b, 0, t)),
          pl.BlockSpec((1, _OUT, _TN), lambda b, t: (b, 0, t)),
      ],
      out_shape=[
          jax.ShapeDtypeStruct((_B, _OUT, _N), _F32),
          jax.ShapeDtypeStruct((_B, _OUT, _N), _F32),
      ],
      compiler_params=pltpu.CompilerParams(
          dimension_semantics=("parallel", "parallel")),
  )(x, x, W1, b1c, W2, b2c, W3, b3c)

  return jnp.stack([out0, out1], axis=-1).reshape(_B, _OUT, _N * _SR)


# speedup vs baseline: 29.6365x; 1.7780x over previous
import functools

import jax
import jax.numpy as jnp
from jax.experimental import pallas as pl
from jax.experimental.pallas import tpu as pltpu

_B, _C, _N = 8, 32, 2048
_OUT, _SR, _K = 64, 2, 4
_TN = 1024
_NT = _N // _TN

_F32 = jnp.float32


def _body(x_full_ref, x_tile_ref, W1_ref, b1_ref, W2_ref, b2_ref,
          W3_ref, b3_ref, out0_ref, out1_ref):
  xb = x_full_ref[0]
  xt = x_tile_ref[0]

  col_n2 = jnp.sum(xb * xb, axis=0, keepdims=True)
  row_n2 = jnp.sum(xt * xt, axis=0)[:, None]
  prod = jax.lax.dot_general(xt, xb, (((0,), (0,)), ((), ())),
                             preferred_element_type=_F32)
  inner = -2.0 * prod
  dist = -col_n2 - inner - row_n2

  iota = jax.lax.broadcasted_iota(jnp.int32, (_TN, _N), 1)

  acc0 = None
  acc1 = None
  for _ in range(_K):
    idx = jnp.argmax(dist, axis=1)[:, None]
    sel = iota == idx
    onehot = sel.astype(_F32)
    xj = jax.lax.dot_general(xb, onehot, (((1,), (1,)), ((), ())),
                             preferred_element_type=_F32)
    dist = jnp.where(sel, -jnp.inf, dist)

    feat = jnp.concatenate([xt, xj], axis=0)
    e1 = jax.lax.dot_general(W1_ref[...], feat, (((1,), (0,)), ((), ())),
                             preferred_element_type=_F32)
    e1 = e1 + b1_ref[...]
    g = jax.nn.relu(jnp.concatenate([e1, feat], axis=0))
    e2 = jax.lax.dot_general(W2_ref[...], g, (((1,), (0,)), ((), ())),
                             preferred_element_type=_F32)
    e2 = jax.nn.relu(e2 + b2_ref[...])
    h0 = jax.lax.dot_general(W3_ref[...], e2[:_OUT], (((1,), (0,)), ((), ())),
                             preferred_element_type=_F32)
    h1 = jax.lax.dot_general(W3_ref[...], e2[_OUT:], (((1,), (0,)), ((), ())),
                             preferred_element_type=_F32)
    h0 = h0 + b3_ref[...]
    h1 = h1 + b3_ref[...]
    acc0 = h0 if acc0 is None else jnp.maximum(acc0, h0)
    acc1 = h1 if acc1 is None else jnp.maximum(acc1, h1)

  out0_ref[0] = acc0
  out1_ref[0] = acc1


@jax.jit
def kernel(x, W1, b1, W2, b2, W3, b3):
  b1c = b1.reshape(_OUT, 1)
  b2c = b2.reshape(_OUT * _SR, 1)
  b3c = b3.reshape(_OUT, 1)

  out0, out1 = pl.pallas_call(
      _body,
      grid=(_B, _NT),
      in_specs=[
          pl.BlockSpec((1, _C, _N), lambda b, t: (b, 0, 0)),
          pl.BlockSpec((1, _C, _TN), lambda b, t: (b, 0, t)),
          pl.BlockSpec((_OUT, 2 * _C), lambda b, t: (0, 0)),
          pl.BlockSpec((_OUT, 1), lambda b, t: (0, 0)),
          pl.BlockSpec((_OUT * _SR, 2 * _C + _OUT), lambda b, t: (0, 0)),
          pl.BlockSpec((_OUT * _SR, 1), lambda b, t: (0, 0)),
          pl.BlockSpec((_OUT, _OUT), lambda b, t: (0, 0)),
          pl.BlockSpec((_OUT, 1), lambda b, t: (0, 0)),
      ],
      out_specs=[
          pl.BlockSpec((1, _OUT, _TN), lambda b, t: (b, 0, t)),
          pl.BlockSpec((1, _OUT, _TN), lambda b, t: (b, 0, t)),
      ],
      out_shape=[
          jax.ShapeDtypeStruct((_B, _OUT, _N), _F32),
          jax.ShapeDtypeStruct((_B, _OUT, _N), _F32),
      ],
      compiler_params=pltpu.CompilerParams(
          dimension_semantics=("parallel", "parallel")),
  )(x, x, W1, b1c, W2, b2c, W3, b3c)

  return jnp.stack([out0, out1], axis=-1).reshape(_B, _OUT, _N * _SR)
